# baseline (speedup 1.0000x reference)
import jax
import jax.numpy as jnp
from jax import lax
from jax.experimental import pallas as pl
from jax.experimental.pallas import tpu as pltpu
from jax.experimental.pallas import tpu_sc as plsc

VOCAB = 100000
DIM = 300
DIMP = 304
B = 1024
L = 50

NW = 32
TOTAL = B * L
CHUNK = 40
CH_PER_W = TOTAL // (NW * CHUNK)
ROWS_PER_W = TOTAL // NW


def _body(table, src_idx, tgt_idx, out_src, out_tgt, idx_v, rows_v, sem):
    wid = lax.axis_index("s") * 2 + lax.axis_index("c")

    def run(idx_hbm, out):
        def body(j, carry):
            base = wid * ROWS_PER_W + j * CHUNK
            pltpu.sync_copy(idx_hbm.at[pl.ds(base, CHUNK)], idx_v)
            pltpu.async_copy(table.at[idx_v], rows_v, sem).wait()
            pltpu.sync_copy(rows_v, out.at[pl.ds(base, CHUNK)])
            return carry
        lax.fori_loop(0, CH_PER_W, body, 0, unroll=False)

    run(src_idx, out_src)
    run(tgt_idx, out_tgt)


_gather_kernel = pl.kernel(
    _body,
    out_type=(
        jax.ShapeDtypeStruct((TOTAL, DIMP), jnp.float32),
        jax.ShapeDtypeStruct((TOTAL, DIMP), jnp.float32),
    ),
    mesh=plsc.VectorSubcoreMesh(core_axis_name="c", subcore_axis_name="s"),
    compiler_params=pltpu.CompilerParams(use_tc_tiling_on_sc=False),
    scratch_types=[
        pltpu.VMEM((CHUNK,), jnp.int32),
        pltpu.VMEM((CHUNK, DIMP), jnp.float32),
        pltpu.SemaphoreType.DMA,
    ],
)


def kernel(src, tgt, embedding_matrix):
    table_p = jnp.pad(embedding_matrix, ((0, 0), (0, DIMP - DIM)))
    out_src, out_tgt = _gather_kernel(
        table_p, src.reshape(TOTAL), tgt.reshape(TOTAL))
    return (out_src[:, :DIM].reshape(B, L, DIM),
            out_tgt[:, :DIM].reshape(B, L, DIM))
